# Initial kernel scaffold; baseline (speedup 1.0000x reference)
import jax
import jax.numpy as jnp
from jax.experimental import pallas as pl


def kernel(x, psm_single, record_len, pairwise_t_matrix):
    raise NotImplementedError("write your pallas kernel here")


# speedup vs baseline: 6.2240x; 6.2240x over previous
import math

import jax
import jax.numpy as jnp
import numpy as np
from jax.experimental import pallas as pl
from jax.experimental.pallas import tpu as pltpu

_TAPS = tuple(
    float(np.exp(-(d * d) / 2.0) / np.sqrt(2.0 * np.pi)) for d in (-2, -1, 0, 1, 2)
)

_K_FRAC = 2


def _conf_kernel(psm_ref, conf_ref, thr_ref):
    p = psm_ref[0]
    m = jnp.max(jax.nn.sigmoid(p), axis=0)
    H, W = m.shape
    zw = jnp.zeros((H, 2), m.dtype)
    pw = jnp.concatenate([zw, m, zw], axis=1)
    acc = _TAPS[0] * pw[:, 0:W]
    for d in range(1, 5):
        acc = acc + _TAPS[d] * pw[:, d:d + W]
    zh = jnp.zeros((2, W), m.dtype)
    ph = jnp.concatenate([zh, acc, zh], axis=0)
    conf = _TAPS[0] * ph[0:H, :]
    for d in range(1, 5):
        conf = conf + _TAPS[d] * ph[d:d + H, :]
    conf_ref[0] = conf

    bits = jax.lax.bitcast_convert_type(conf, jnp.int32)
    k = (H * W) // _K_FRAC

    def body(_, lohi):
        lo, hi = lohi
        mid = (lo + hi) // 2
        c = jnp.sum((bits >= mid).astype(jnp.int32))
        take = c >= k
        return jnp.where(take, mid, lo), jnp.where(take, hi, mid)

    lo, _ = jax.lax.fori_loop(
        0, 30, body, (jnp.int32(0), jnp.int32(1 << 30)), unroll=False
    )
    thr_ref[0, 0, 0] = jax.lax.bitcast_convert_type(lo, jnp.float32)


def _fuse_kernel(thr_ref, x_ref, conf_ref, out_ref):
    xb = x_ref[0]
    cb = conf_ref[0]
    L, C, BLK = xb.shape
    thr = jnp.stack([thr_ref[0, 0, i] for i in range(L)])[:, None]
    row = jax.lax.broadcasted_iota(jnp.int32, (L, BLK), 0)
    mask = (cb >= thr) | (row == 0)
    dots = jnp.sum(xb * xb[0:1], axis=1) * (1.0 / math.sqrt(C))
    s = jnp.where(mask, dots, 0.0)
    e = jnp.exp(s - jnp.max(s, axis=0, keepdims=True))
    w = jnp.where(mask, e, 0.0) / jnp.sum(e, axis=0, keepdims=True)
    out_ref[0] = jnp.sum(w[:, None, :] * xb, axis=0)


def kernel(x, psm_single, record_len, pairwise_t_matrix):
    B = record_len.shape[0]
    N, C, H, W = x.shape
    L = N // B
    A = psm_single.shape[1]
    HW = H * W

    conf, thr = pl.pallas_call(
        _conf_kernel,
        grid=(N,),
        in_specs=[pl.BlockSpec((1, A, H, W), lambda i: (i, 0, 0, 0))],
        out_specs=[
            pl.BlockSpec((1, H, W), lambda i: (i, 0, 0)),
            pl.BlockSpec((1, 1, 1), lambda i: (i, 0, 0), memory_space=pltpu.SMEM),
        ],
        out_shape=[
            jax.ShapeDtypeStruct((N, H, W), jnp.float32),
            jax.ShapeDtypeStruct((N, 1, 1), jnp.float32),
        ],
    )(psm_single)

    BLK = 512
    xr = x.reshape(B, L, C, HW)
    cr = conf.reshape(B, L, HW)
    tr = thr.reshape(B, 1, L)

    fused = pl.pallas_call(
        _fuse_kernel,
        grid=(B, HW // BLK),
        in_specs=[
            pl.BlockSpec((1, 1, L), lambda b, j: (b, 0, 0), memory_space=pltpu.SMEM),
            pl.BlockSpec((1, L, C, BLK), lambda b, j: (b, 0, 0, j)),
            pl.BlockSpec((1, L, BLK), lambda b, j: (b, 0, j)),
        ],
        out_specs=pl.BlockSpec((1, C, BLK), lambda b, j: (b, 0, j)),
        out_shape=jax.ShapeDtypeStruct((B, C, HW), jnp.float32),
    )(tr, xr, cr)

    fused = fused.reshape(B, C, H, W)
    rate = jnp.float32(1.0 / _K_FRAC)
    return fused, rate


# speedup vs baseline: 7.8697x; 1.2644x over previous
import math

import jax
import jax.numpy as jnp
import numpy as np
from jax.experimental import pallas as pl
from jax.experimental.pallas import tpu as pltpu

_TAPS = tuple(
    float(np.exp(-(d * d) / 2.0) / np.sqrt(2.0 * np.pi)) for d in (-2, -1, 0, 1, 2)
)

_K_FRAC = 2


def _conf_kernel(psm_ref, conf_ref, thr_ref):
    p = psm_ref[...]
    m = jnp.max(jax.nn.sigmoid(p), axis=1)
    N, H, W = m.shape
    zw = jnp.zeros((N, H, 2), m.dtype)
    pw = jnp.concatenate([zw, m, zw], axis=2)
    acc = _TAPS[0] * pw[:, :, 0:W]
    for d in range(1, 5):
        acc = acc + _TAPS[d] * pw[:, :, d:d + W]
    zh = jnp.zeros((N, 2, W), m.dtype)
    ph = jnp.concatenate([zh, acc, zh], axis=1)
    conf = _TAPS[0] * ph[:, 0:H, :]
    for d in range(1, 5):
        conf = conf + _TAPS[d] * ph[:, d:d + H, :]
    conf_ref[...] = conf

    bits = jax.lax.bitcast_convert_type(conf, jnp.int32)
    k = (H * W) // _K_FRAC

    def body(_, lohi):
        lo, hi = lohi
        mid = (lo + hi) // 2
        c = jnp.sum((bits >= mid).astype(jnp.int32), axis=(1, 2), keepdims=True)
        take = c >= k
        return jnp.where(take, mid, lo), jnp.where(take, hi, mid)

    init = (
        jnp.zeros((N, 1, 1), jnp.int32),
        jnp.full((N, 1, 1), 1 << 30, jnp.int32),
    )
    lo, _ = jax.lax.fori_loop(0, 30, body, init, unroll=False)
    thr_ref[...] = jax.lax.bitcast_convert_type(lo, jnp.float32)


def _fuse_kernel(thr_ref, x_ref, conf_ref, out_ref):
    xb = x_ref[0]
    cb = conf_ref[0]
    L, C, BLK = xb.shape
    thr = jnp.stack([thr_ref[0, 0, i] for i in range(L)])[:, None]
    row = jax.lax.broadcasted_iota(jnp.int32, (L, BLK), 0)
    mask = (cb >= thr) | (row == 0)
    dots = jnp.sum(xb * xb[0:1], axis=1) * (1.0 / math.sqrt(C))
    s = jnp.where(mask, dots, 0.0)
    e = jnp.exp(s - jnp.max(s, axis=0, keepdims=True))
    w = jnp.where(mask, e, 0.0) / jnp.sum(e, axis=0, keepdims=True)
    out_ref[0] = jnp.sum(w[:, None, :] * xb, axis=0)


def kernel(x, psm_single, record_len, pairwise_t_matrix):
    B = record_len.shape[0]
    N, C, H, W = x.shape
    L = N // B
    A = psm_single.shape[1]
    HW = H * W

    conf, thr = pl.pallas_call(
        _conf_kernel,
        in_specs=[pl.BlockSpec((N, A, H, W), lambda: (0, 0, 0, 0))],
        out_specs=[
            pl.BlockSpec((N, H, W), lambda: (0, 0, 0)),
            pl.BlockSpec((N, 1, 1), lambda: (0, 0, 0)),
        ],
        out_shape=[
            jax.ShapeDtypeStruct((N, H, W), jnp.float32),
            jax.ShapeDtypeStruct((N, 1, 1), jnp.float32),
        ],
    )(psm_single)

    BLK = 2048
    xr = x.reshape(B, L, C, HW)
    cr = conf.reshape(B, L, HW)
    tr = thr.reshape(B, 1, L)

    fused = pl.pallas_call(
        _fuse_kernel,
        grid=(B, HW // BLK),
        in_specs=[
            pl.BlockSpec((1, 1, L), lambda b, j: (b, 0, 0), memory_space=pltpu.SMEM),
            pl.BlockSpec((1, L, C, BLK), lambda b, j: (b, 0, 0, j)),
            pl.BlockSpec((1, L, BLK), lambda b, j: (b, 0, j)),
        ],
        out_specs=pl.BlockSpec((1, C, BLK), lambda b, j: (b, 0, j)),
        out_shape=jax.ShapeDtypeStruct((B, C, HW), jnp.float32),
    )(tr, xr, cr)

    fused = fused.reshape(B, C, H, W)
    rate = jnp.float32(1.0 / _K_FRAC)
    return fused, rate
